# baseline (speedup 1.0000x reference)
import functools

import jax
import jax.numpy as jnp
from jax import lax
from jax.experimental import pallas as pl
from jax.experimental.pallas import tpu as pltpu

H = 2048
E = 64
K = 8
_NEG = -1e30


def _gate_body(x_ref, wn_ref, wt_ref, noise_ref, gates_ref, idx_ref,
               acc_sp, acc_x, *, n_tokens):
    i = pl.program_id(0)

    @pl.when(i == 0)
    def _init():
        acc_sp[...] = jnp.zeros_like(acc_sp)
        acc_x[...] = jnp.zeros_like(acc_x)

    xb = x_ref[...]
    g = lax.dot_general(xb, wn_ref[...], (((1,), (1,)), ((), ())),
                        preferred_element_type=jnp.float32)
    acc_sp[...] += jnp.sum(jax.nn.softplus(g), axis=0, keepdims=True)
    acc_x[...] += jnp.sum(xb, axis=0, keepdims=True)

    @pl.when(i == pl.num_programs(0) - 1)
    def _finish():
        ninv = jnp.float32(1.0 / n_tokens)
        mean_t = lax.dot_general(acc_x[...] * ninv, wt_ref[...],
                                 (((1,), (1,)), ((), ())),
                                 preferred_element_type=jnp.float32)
        mw = mean_t + acc_sp[...] * ninv * noise_ref[...]

        iota = lax.broadcasted_iota(jnp.int32, (1, E), 1)
        iota_k = lax.broadcasted_iota(jnp.int32, (1, K), 1)
        vals = mw
        gout = jnp.zeros((1, K), jnp.float32)
        iout = jnp.zeros((1, K), jnp.int32)
        g0 = jnp.float32(0.0)
        for k in range(K):
            m = jnp.max(vals)
            if k == 0:
                g0 = m
            sel = jnp.min(jnp.where(vals == m, iota, E))
            gout = jnp.where(iota_k == k, m, gout)
            iout = jnp.where(iota_k == k, sel, iout)
            vals = jnp.where(iota == sel, _NEG, vals)
        e = jnp.exp(gout - g0)
        gates_ref[...] = e / jnp.sum(e)
        idx_ref[...] = iout


def kernel(x, W_transform, W_noise):
    n_tokens = x.shape[0] * x.shape[1]
    x2d = x.reshape(n_tokens, H)
    noise = jax.random.normal(jax.random.key(42), (E,), dtype=x.dtype)
    noise2d = noise.reshape(1, E)

    bt = 2048
    grid = (n_tokens // bt,)
    gates, idx = pl.pallas_call(
        functools.partial(_gate_body, n_tokens=n_tokens),
        grid=grid,
        in_specs=[
            pl.BlockSpec((bt, H), lambda i: (i, 0)),
            pl.BlockSpec((E, H), lambda i: (0, 0)),
            pl.BlockSpec((E, H), lambda i: (0, 0)),
            pl.BlockSpec((1, E), lambda i: (0, 0)),
        ],
        out_specs=[
            pl.BlockSpec((1, K), lambda i: (0, 0)),
            pl.BlockSpec((1, K), lambda i: (0, 0)),
        ],
        out_shape=[
            jax.ShapeDtypeStruct((1, K), jnp.float32),
            jax.ShapeDtypeStruct((1, K), jnp.int32),
        ],
        scratch_shapes=[
            pltpu.VMEM((1, E), jnp.float32),
            pltpu.VMEM((1, H), jnp.float32),
        ],
    )(x2d, W_noise, W_transform, noise2d)
    return gates.reshape(K), idx.reshape(K)


# speedup vs baseline: 1.0705x; 1.0705x over previous
import functools

import jax
import jax.numpy as jnp
from jax import lax
from jax.experimental import pallas as pl
from jax.experimental.pallas import tpu as pltpu

H = 2048
E = 64
K = 8
_NEG = -1e30


def _gate_body(x_ref, wn_ref, wt_ref, noise_ref, gates_ref, idx_ref,
               acc_sp, acc_x, *, n_tokens):
    i = pl.program_id(0)

    @pl.when(i == 0)
    def _init():
        acc_sp[...] = jnp.zeros_like(acc_sp)
        acc_x[...] = jnp.zeros_like(acc_x)

    xb = x_ref[...]
    g = lax.dot_general(xb, wn_ref[...], (((1,), (1,)), ((), ())),
                        preferred_element_type=jnp.float32)
    acc_sp[...] += jnp.sum(jax.nn.softplus(g), axis=0, keepdims=True)
    acc_x[...] += jnp.sum(xb, axis=0, keepdims=True)

    @pl.when(i == pl.num_programs(0) - 1)
    def _finish():
        ninv = jnp.float32(1.0 / n_tokens)
        mean_t = lax.dot_general(acc_x[...] * ninv, wt_ref[...],
                                 (((1,), (1,)), ((), ())),
                                 preferred_element_type=jnp.float32)
        mw = mean_t + acc_sp[...] * ninv * noise_ref[...]

        iota = lax.broadcasted_iota(jnp.int32, (1, E), 1)
        iota_k = lax.broadcasted_iota(jnp.int32, (1, K), 1)
        vals = mw
        gout = jnp.zeros((1, K), jnp.float32)
        iout = jnp.zeros((1, K), jnp.int32)
        g0 = jnp.float32(0.0)
        for k in range(K):
            m = jnp.max(vals)
            if k == 0:
                g0 = m
            sel = jnp.min(jnp.where(vals == m, iota, E))
            gout = jnp.where(iota_k == k, m, gout)
            iout = jnp.where(iota_k == k, sel, iout)
            vals = jnp.where(iota == sel, _NEG, vals)
        e = jnp.exp(gout - g0)
        gates_ref[...] = e / jnp.sum(e)
        idx_ref[...] = iout


def kernel(x, W_transform, W_noise):
    n_tokens = x.shape[0] * x.shape[1]
    x2d = x.reshape(n_tokens, H)
    noise = jax.random.normal(jax.random.key(42), (E,), dtype=x.dtype)
    noise2d = noise.reshape(1, E)

    bt = 1024
    grid = (n_tokens // bt,)
    gates, idx = pl.pallas_call(
        functools.partial(_gate_body, n_tokens=n_tokens),
        grid=grid,
        in_specs=[
            pl.BlockSpec((bt, H), lambda i: (i, 0)),
            pl.BlockSpec((E, H), lambda i: (0, 0)),
            pl.BlockSpec((E, H), lambda i: (0, 0)),
            pl.BlockSpec((1, E), lambda i: (0, 0)),
        ],
        out_specs=[
            pl.BlockSpec((1, K), lambda i: (0, 0)),
            pl.BlockSpec((1, K), lambda i: (0, 0)),
        ],
        out_shape=[
            jax.ShapeDtypeStruct((1, K), jnp.float32),
            jax.ShapeDtypeStruct((1, K), jnp.int32),
        ],
        scratch_shapes=[
            pltpu.VMEM((1, E), jnp.float32),
            pltpu.VMEM((1, H), jnp.float32),
        ],
    )(x2d, W_noise, W_transform, noise2d)
    return gates.reshape(K), idx.reshape(K)
